# baseline (speedup 1.0000x reference)
import jax
import jax.numpy as jnp
from jax.experimental import pallas as pl
from jax.experimental.pallas import tpu as pltpu

_N = 67108864
_NBUF = 6
_CHUNK = 2 * 1024 * 1024
_NCHUNK = _N // _CHUNK
_DEPTH = 3


def _copy_kernel(x_hbm, o_hbm, *scratch):
    bufs, isem, osem = scratch[:_NBUF], scratch[_NBUF], scratch[_NBUF + 1]

    def in_copy(i):
        return pltpu.make_async_copy(
            x_hbm.at[pl.ds(i * _CHUNK, _CHUNK)], bufs[i % _NBUF],
            isem.at[i % _NBUF])

    def out_copy(i):
        return pltpu.make_async_copy(
            bufs[i % _NBUF], o_hbm.at[pl.ds(i * _CHUNK, _CHUNK)],
            osem.at[i % _NBUF])

    for j in range(_DEPTH):
        in_copy(j).start()
    for i in range(_NCHUNK):
        j = i + _DEPTH
        if j < _NCHUNK:
            if j >= _NBUF:
                out_copy(j - _NBUF).wait()
            in_copy(j).start()
        in_copy(i).wait()
        if i == 0:
            buf = bufs[0]
            idx = jax.lax.broadcasted_iota(jnp.int32, (1024,), 0)
            buf[0:1024] = jnp.where(idx == 0, jnp.float32(0.0), buf[0:1024])
        out_copy(i).start()
    for i in range(_NCHUNK - _NBUF, _NCHUNK):
        out_copy(i).wait()


def kernel(x):
    return pl.pallas_call(
        _copy_kernel,
        in_specs=[pl.BlockSpec(memory_space=pl.ANY)],
        out_specs=pl.BlockSpec(memory_space=pl.ANY),
        out_shape=jax.ShapeDtypeStruct((_N,), x.dtype),
        scratch_shapes=(
            [pltpu.VMEM((_CHUNK,), jnp.float32) for _ in range(_NBUF)]
            + [pltpu.SemaphoreType.DMA((_NBUF,)),
               pltpu.SemaphoreType.DMA((_NBUF,))]
        ),
    )(x)


# speedup vs baseline: 1.0069x; 1.0069x over previous
import jax
import jax.numpy as jnp
from jax.experimental import pallas as pl
from jax.experimental.pallas import tpu as pltpu

_N = 67108864
_NBUF = 3
_CHUNK = 4 * 1024 * 1024
_NCHUNK = _N // _CHUNK
_DEPTH = 2


def _copy_kernel(x_hbm, o_hbm, *scratch):
    bufs, isem, osem = scratch[:_NBUF], scratch[_NBUF], scratch[_NBUF + 1]

    def in_copy(i):
        return pltpu.make_async_copy(
            x_hbm.at[pl.ds(i * _CHUNK, _CHUNK)], bufs[i % _NBUF],
            isem.at[i % _NBUF])

    def out_copy(i):
        return pltpu.make_async_copy(
            bufs[i % _NBUF], o_hbm.at[pl.ds(i * _CHUNK, _CHUNK)],
            osem.at[i % _NBUF])

    for j in range(_DEPTH):
        in_copy(j).start()
    for i in range(_NCHUNK):
        j = i + _DEPTH
        if j < _NCHUNK:
            if j >= _NBUF:
                out_copy(j - _NBUF).wait()
            in_copy(j).start()
        in_copy(i).wait()
        if i == 0:
            buf = bufs[0]
            idx = jax.lax.broadcasted_iota(jnp.int32, (1024,), 0)
            buf[0:1024] = jnp.where(idx == 0, jnp.float32(0.0), buf[0:1024])
        out_copy(i).start()
    for i in range(_NCHUNK - _NBUF, _NCHUNK):
        out_copy(i).wait()


def kernel(x):
    return pl.pallas_call(
        _copy_kernel,
        in_specs=[pl.BlockSpec(memory_space=pl.ANY)],
        out_specs=pl.BlockSpec(memory_space=pl.ANY),
        out_shape=jax.ShapeDtypeStruct((_N,), x.dtype),
        scratch_shapes=(
            [pltpu.VMEM((_CHUNK,), jnp.float32) for _ in range(_NBUF)]
            + [pltpu.SemaphoreType.DMA((_NBUF,)),
               pltpu.SemaphoreType.DMA((_NBUF,))]
        ),
    )(x)
